# baseline (speedup 1.0000x reference)
import functools

import jax
import jax.numpy as jnp
from jax import lax
from jax.experimental import pallas as pl
from jax.experimental.pallas import tpu as pltpu
from jax.experimental.pallas import tpu_sc as plsc

N_HITS = 1048576
N_VERTS = 262144
N_TRIS = 524288
N_PARAMS = 16

_info = plsc.get_sparse_core_info()
_NC, _NS, _L = _info.num_cores, _info.num_subcores, _info.num_lanes
_NW = _NC * _NS

CHUNK = 1024
HITS_PER_W = N_HITS // _NW
CHUNKS_PER_W = HITS_PER_W // CHUNK
GROUPS = CHUNK // _L
_SL = 128
_NSL = CHUNK // _SL


def _body(hpf_hbm, prim_hbm, vbof_hbm, ibof_hbm, feat_hbm, out_hbm,
          prim_v, idx_v, w_v, hp_v, vc_v, f0_v, f1_v, f2_v, out_v, sem):
    wid = lax.axis_index("s") * _NC + lax.axis_index("c")
    wbase = wid * HITS_PER_W

    def chunk_body(g, carry):
        base = wbase + g * CHUNK

        pltpu.sync_copy(prim_hbm.at[pl.ds(base, CHUNK)], prim_v)

        def tri_idx_body(j, c):
            sl = pl.ds(j * _L, _L)
            p3 = prim_v[sl] * 3
            w_v[0, sl] = p3
            w_v[1, sl] = p3 + 1
            w_v[2, sl] = p3 + 2
            return c
        lax.fori_loop(0, GROUPS, tri_idx_body, 0)

        cps = [
            pltpu.async_copy(
                ibof_hbm.at[w_v.at[k, pl.ds(s * _SL, _SL)]],
                idx_v.at[k, pl.ds(s * _SL, _SL)], sem)
            for k in range(3)
            for s in range(_NSL)
        ]
        for c in cps:
            c.wait()

        fbufs = (f0_v, f1_v, f2_v)
        cps = [
            pltpu.async_copy(
                feat_hbm.at[idx_v.at[k, pl.ds(s * _SL, _SL)]],
                fbufs[k].at[pl.ds(s * _SL, _SL)], sem)
            for k in range(3)
            for s in range(_NSL)
        ]
        cps.append(
            pltpu.async_copy(hpf_hbm.at[pl.ds(3 * base, 3 * CHUNK)],
                             hp_v, sem))

        def vtx_idx_body(j, c):
            sl = pl.ds(j * _L, _L)
            for k in range(3):
                v3 = idx_v[k, sl] * 3
                w_v[3 * k, sl] = v3
                w_v[3 * k + 1, sl] = v3 + 1
                w_v[3 * k + 2, sl] = v3 + 2
            return c
        lax.fori_loop(0, GROUPS, vtx_idx_body, 0)

        cps += [
            pltpu.async_copy(
                vbof_hbm.at[w_v.at[r, pl.ds(s * _SL, _SL)]],
                vc_v.at[r, pl.ds(s * _SL, _SL)], sem)
            for r in range(9)
            for s in range(_NSL)
        ]
        for c in cps:
            c.wait()

        def grp(j, c):
            b = j * _L
            sl = pl.ds(b, _L)
            i3 = (b + lax.iota(jnp.int32, _L)) * 3
            hx = plsc.load_gather(hp_v, [i3])
            hy = plsc.load_gather(hp_v, [i3 + 1])
            hz = plsc.load_gather(hp_v, [i3 + 2])
            ax, ay, az = vc_v[0, sl], vc_v[1, sl], vc_v[2, sl]
            bx, by, bz = vc_v[3, sl], vc_v[4, sl], vc_v[5, sl]
            cx, cy, cz = vc_v[6, sl], vc_v[7, sl], vc_v[8, sl]

            e0x, e0y, e0z = bx - ax, by - ay, bz - az
            e1x, e1y, e1z = cx - ax, cy - ay, cz - az
            px, py, pz = hx - ax, hy - ay, hz - az

            d00 = e0x * e0x + e0y * e0y + e0z * e0z
            d01 = e0x * e1x + e0y * e1y + e0z * e1z
            d11 = e1x * e1x + e1y * e1y + e1z * e1z
            d20 = px * e0x + py * e0y + pz * e0z
            d21 = px * e1x + py * e1y + pz * e1z

            denom = d00 * d11 - d01 * d01
            denom = jnp.where(jnp.abs(denom) < 1e-12,
                              jnp.float32(1e-12), denom)
            rec = 1.0 / denom
            vv = (d11 * d20 - d01 * d21) * rec
            ww = (d00 * d21 - d01 * d20) * rec
            uu = 1.0 - vv - ww

            for i in range(_L):
                out_v[b + i, :] = (uu[i] * f0_v[b + i, :]
                                   + vv[i] * f1_v[b + i, :]
                                   + ww[i] * f2_v[b + i, :])
            return c
        lax.fori_loop(0, GROUPS, grp, 0)

        pltpu.sync_copy(out_v, out_hbm.at[pl.ds(base, CHUNK)])
        return carry

    lax.fori_loop(0, CHUNKS_PER_W, chunk_body, 0)


_sc_kernel = functools.partial(
    pl.kernel,
    out_type=jax.ShapeDtypeStruct((N_HITS, N_PARAMS), jnp.float32),
    mesh=plsc.VectorSubcoreMesh(core_axis_name="c", subcore_axis_name="s"),
    scratch_types=[
        pltpu.VMEM((CHUNK,), jnp.int32),
        pltpu.VMEM((3, CHUNK), jnp.int32),
        pltpu.VMEM((9, CHUNK), jnp.int32),
        pltpu.VMEM((3 * CHUNK,), jnp.float32),
        pltpu.VMEM((9, CHUNK), jnp.float32),
        pltpu.VMEM((CHUNK, N_PARAMS), jnp.float32),
        pltpu.VMEM((CHUNK, N_PARAMS), jnp.float32),
        pltpu.VMEM((CHUNK, N_PARAMS), jnp.float32),
        pltpu.VMEM((CHUNK, N_PARAMS), jnp.float32),
        pltpu.SemaphoreType.DMA,
    ],
    compiler_params=pltpu.CompilerParams(
        use_tc_tiling_on_sc=False,
        needs_layout_passes=False,
    ),
)(_body)


@jax.jit
def kernel(hit_positions, hit_primIDs, vbo, ibo, features):
    hp_f = hit_positions.reshape(-1)
    vbo_f = vbo.reshape(-1)
    ibo_f = ibo.reshape(-1)
    return _sc_kernel(hp_f, hit_primIDs, vbo_f, ibo_f, features)


# speedup vs baseline: 2.5436x; 2.5436x over previous
import functools

import jax
import jax.numpy as jnp
from jax import lax
from jax.experimental import pallas as pl
from jax.experimental.pallas import tpu as pltpu
from jax.experimental.pallas import tpu_sc as plsc

N_HITS = 1048576
N_VERTS = 262144
N_TRIS = 524288
N_PARAMS = 16

_info = plsc.get_sparse_core_info()
_NC, _NS, _L = _info.num_cores, _info.num_subcores, _info.num_lanes
_NW = _NC * _NS

CHUNK = 1024
HITS_PER_W = N_HITS // _NW
CHUNKS_PER_W = HITS_PER_W // CHUNK
GROUPS = CHUNK // _L
_SL = 128
_NSL = CHUNK // _SL


def _body(hx_hbm, hy_hbm, hz_hbm, prim_hbm,
          i0_hbm, i1_hbm, i2_hbm, vx_hbm, vy_hbm, vz_hbm, feat_hbm,
          out_hbm,
          prim_v, idx_v, hp_v, vc_v, f0_v, f1_v, f2_v, out_v, sem):
    wid = lax.axis_index("s") * _NC + lax.axis_index("c")
    wbase = wid * HITS_PER_W

    def chunk_body(g, carry):
        base = wbase + g * CHUNK

        pltpu.sync_copy(prim_hbm.at[pl.ds(base, CHUNK)], prim_v)
        ibufs = (i0_hbm, i1_hbm, i2_hbm)
        cps = [
            pltpu.async_copy(
                ibufs[k].at[prim_v.at[pl.ds(s * _SL, _SL)]],
                idx_v.at[k, pl.ds(s * _SL, _SL)], sem)
            for k in range(3)
            for s in range(_NSL)
        ]
        for c in cps:
            c.wait()

        cps = [
            pltpu.async_copy(hx_hbm.at[pl.ds(base, CHUNK)],
                             hp_v.at[0], sem),
            pltpu.async_copy(hy_hbm.at[pl.ds(base, CHUNK)],
                             hp_v.at[1], sem),
            pltpu.async_copy(hz_hbm.at[pl.ds(base, CHUNK)],
                             hp_v.at[2], sem),
        ]
        vbufs = (vx_hbm, vy_hbm, vz_hbm)
        fbufs = (f0_v, f1_v, f2_v)
        for k in range(3):
            for s in range(_NSL):
                isl = idx_v.at[k, pl.ds(s * _SL, _SL)]
                sl = pl.ds(s * _SL, _SL)
                cps += [
                    pltpu.async_copy(vbufs[d].at[isl],
                                     vc_v.at[3 * k + d, sl], sem)
                    for d in range(3)
                ]
                cps.append(
                    pltpu.async_copy(feat_hbm.at[isl], fbufs[k].at[sl],
                                     sem))
        for c in cps:
            c.wait()

        def grp(j, c):
            b = j * _L
            sl = pl.ds(b, _L)
            hx, hy, hz = hp_v[0, sl], hp_v[1, sl], hp_v[2, sl]
            ax, ay, az = vc_v[0, sl], vc_v[1, sl], vc_v[2, sl]
            bx, by, bz = vc_v[3, sl], vc_v[4, sl], vc_v[5, sl]
            cx, cy, cz = vc_v[6, sl], vc_v[7, sl], vc_v[8, sl]

            e0x, e0y, e0z = bx - ax, by - ay, bz - az
            e1x, e1y, e1z = cx - ax, cy - ay, cz - az
            px, py, pz = hx - ax, hy - ay, hz - az

            d00 = e0x * e0x + e0y * e0y + e0z * e0z
            d01 = e0x * e1x + e0y * e1y + e0z * e1z
            d11 = e1x * e1x + e1y * e1y + e1z * e1z
            d20 = px * e0x + py * e0y + pz * e0z
            d21 = px * e1x + py * e1y + pz * e1z

            denom = d00 * d11 - d01 * d01
            denom = jnp.where(jnp.abs(denom) < 1e-12,
                              jnp.float32(1e-12), denom)
            rec = 1.0 / denom
            vv = (d11 * d20 - d01 * d21) * rec
            ww = (d00 * d21 - d01 * d20) * rec
            uu = 1.0 - vv - ww

            for i in range(_L):
                out_v[b + i, :] = (uu[i] * f0_v[b + i, :]
                                   + vv[i] * f1_v[b + i, :]
                                   + ww[i] * f2_v[b + i, :])
            return c
        lax.fori_loop(0, GROUPS, grp, 0)

        pltpu.sync_copy(out_v, out_hbm.at[pl.ds(base, CHUNK)])
        return carry

    lax.fori_loop(0, CHUNKS_PER_W, chunk_body, 0)


_sc_kernel = functools.partial(
    pl.kernel,
    out_type=jax.ShapeDtypeStruct((N_HITS, N_PARAMS), jnp.float32),
    mesh=plsc.VectorSubcoreMesh(core_axis_name="c", subcore_axis_name="s"),
    scratch_types=[
        pltpu.VMEM((CHUNK,), jnp.int32),
        pltpu.VMEM((3, CHUNK), jnp.int32),
        pltpu.VMEM((3, CHUNK), jnp.float32),
        pltpu.VMEM((9, CHUNK), jnp.float32),
        pltpu.VMEM((CHUNK, N_PARAMS), jnp.float32),
        pltpu.VMEM((CHUNK, N_PARAMS), jnp.float32),
        pltpu.VMEM((CHUNK, N_PARAMS), jnp.float32),
        pltpu.VMEM((CHUNK, N_PARAMS), jnp.float32),
        pltpu.SemaphoreType.DMA,
    ],
    compiler_params=pltpu.CompilerParams(
        use_tc_tiling_on_sc=False,
        needs_layout_passes=False,
    ),
)(_body)


@jax.jit
def kernel(hit_positions, hit_primIDs, vbo, ibo, features):
    return _sc_kernel(
        hit_positions[:, 0], hit_positions[:, 1], hit_positions[:, 2],
        hit_primIDs,
        ibo[:, 0], ibo[:, 1], ibo[:, 2],
        vbo[:, 0], vbo[:, 1], vbo[:, 2],
        features)


# speedup vs baseline: 2.7243x; 1.0710x over previous
import functools

import jax
import jax.numpy as jnp
from jax import lax
from jax.experimental import pallas as pl
from jax.experimental.pallas import tpu as pltpu
from jax.experimental.pallas import tpu_sc as plsc

N_HITS = 1048576
N_VERTS = 262144
N_TRIS = 524288
N_PARAMS = 16

_info = plsc.get_sparse_core_info()
_NC, _NS, _L = _info.num_cores, _info.num_subcores, _info.num_lanes
_NW = _NC * _NS

CHUNK = 512
HITS_PER_W = N_HITS // _NW
NCH = HITS_PER_W // CHUNK
GROUPS = CHUNK // _L
_SL = 128
_NSL = CHUNK // _SL


def _body(hx_hbm, hy_hbm, hz_hbm, prim_hbm,
          i0_hbm, i1_hbm, i2_hbm, vx_hbm, vy_hbm, vz_hbm, feat_hbm,
          out_hbm,
          prim_v, idx_v, hp_v, vc_v, f0_v, f1_v, f2_v, out_v,
          sem_t, sem_b):
    wid = lax.axis_index("s") * _NC + lax.axis_index("c")
    wbase = wid * HITS_PER_W
    ibufs = (i0_hbm, i1_hbm, i2_hbm)
    vbufs = (vx_hbm, vy_hbm, vz_hbm)
    hbufs = (hx_hbm, hy_hbm, hz_hbm)
    fbufs = (f0_v, f1_v, f2_v)

    def tri_descs(n, r, mk):
        del n
        op = pltpu.make_async_copy if mk else pltpu.async_copy
        return [
            op(ibufs[k].at[prim_v.at[r, pl.ds(s * _SL, _SL)]],
               idx_v.at[r, k, pl.ds(s * _SL, _SL)], sem_t)
            for k in range(3)
            for s in range(_NSL)
        ]

    def prim_load(n, r):
        base = wbase + n * CHUNK
        pltpu.sync_copy(prim_hbm.at[pl.ds(base, CHUNK)], prim_v.at[r])

    def bulk_descs(n, r, mk):
        base = wbase + n * CHUNK
        op = pltpu.make_async_copy if mk else pltpu.async_copy
        cps = [
            op(hbufs[d].at[pl.ds(base, CHUNK)], hp_v.at[r, d], sem_b)
            for d in range(3)
        ]
        for k in range(3):
            for s in range(_NSL):
                isl = idx_v.at[r, k, pl.ds(s * _SL, _SL)]
                sl = pl.ds(s * _SL, _SL)
                cps += [
                    op(vbufs[d].at[isl], vc_v.at[r, 3 * k + d, sl], sem_b)
                    for d in range(3)
                ]
                cps.append(op(feat_hbm.at[isl], fbufs[k].at[r, sl], sem_b))
        return cps

    def compute(n, r):
        def grp(j, c):
            b = j * _L
            sl = pl.ds(b, _L)
            hx, hy, hz = hp_v[r, 0, sl], hp_v[r, 1, sl], hp_v[r, 2, sl]
            ax, ay, az = vc_v[r, 0, sl], vc_v[r, 1, sl], vc_v[r, 2, sl]
            bx, by, bz = vc_v[r, 3, sl], vc_v[r, 4, sl], vc_v[r, 5, sl]
            cx, cy, cz = vc_v[r, 6, sl], vc_v[r, 7, sl], vc_v[r, 8, sl]

            e0x, e0y, e0z = bx - ax, by - ay, bz - az
            e1x, e1y, e1z = cx - ax, cy - ay, cz - az
            px, py, pz = hx - ax, hy - ay, hz - az

            d00 = e0x * e0x + e0y * e0y + e0z * e0z
            d01 = e0x * e1x + e0y * e1y + e0z * e1z
            d11 = e1x * e1x + e1y * e1y + e1z * e1z
            d20 = px * e0x + py * e0y + pz * e0z
            d21 = px * e1x + py * e1y + pz * e1z

            denom = d00 * d11 - d01 * d01
            denom = jnp.where(jnp.abs(denom) < 1e-12,
                              jnp.float32(1e-12), denom)
            rec = 1.0 / denom
            vv = (d11 * d20 - d01 * d21) * rec
            ww = (d00 * d21 - d01 * d20) * rec
            uu = 1.0 - vv - ww

            for i in range(_L):
                out_v[b + i, :] = (uu[i] * f0_v[r, b + i, :]
                                   + vv[i] * f1_v[r, b + i, :]
                                   + ww[i] * f2_v[r, b + i, :])
            return c
        lax.fori_loop(0, GROUPS, grp, 0)
        base = wbase + n * CHUNK
        pltpu.sync_copy(out_v, out_hbm.at[pl.ds(base, CHUNK)])

    prim_load(0, 0)
    tri_descs(0, 0, False)
    for d in tri_descs(0, 0, True):
        d.wait()
    bulk_descs(0, 0, False)
    prim_load(1, 1)
    tri_descs(1, 1, False)

    def chunk_body(g, carry):
        p = g % 2
        q = 1 - p
        for d in bulk_descs(g, p, True):
            d.wait()

        @pl.when(g < NCH - 1)
        def _():
            for d in tri_descs(g + 1, q, True):
                d.wait()
            bulk_descs(g + 1, q, False)

        @pl.when(g < NCH - 2)
        def _():
            prim_load(g + 2, p)
            tri_descs(g + 2, p, False)

        compute(g, p)
        return carry

    lax.fori_loop(0, NCH, chunk_body, 0)


_sc_kernel = functools.partial(
    pl.kernel,
    out_type=jax.ShapeDtypeStruct((N_HITS, N_PARAMS), jnp.float32),
    mesh=plsc.VectorSubcoreMesh(core_axis_name="c", subcore_axis_name="s"),
    scratch_types=[
        pltpu.VMEM((2, CHUNK), jnp.int32),
        pltpu.VMEM((2, 3, CHUNK), jnp.int32),
        pltpu.VMEM((2, 3, CHUNK), jnp.float32),
        pltpu.VMEM((2, 9, CHUNK), jnp.float32),
        pltpu.VMEM((2, CHUNK, N_PARAMS), jnp.float32),
        pltpu.VMEM((2, CHUNK, N_PARAMS), jnp.float32),
        pltpu.VMEM((2, CHUNK, N_PARAMS), jnp.float32),
        pltpu.VMEM((CHUNK, N_PARAMS), jnp.float32),
        pltpu.SemaphoreType.DMA,
        pltpu.SemaphoreType.DMA,
    ],
    compiler_params=pltpu.CompilerParams(
        use_tc_tiling_on_sc=False,
        needs_layout_passes=False,
    ),
)(_body)


@jax.jit
def kernel(hit_positions, hit_primIDs, vbo, ibo, features):
    return _sc_kernel(
        hit_positions[:, 0], hit_positions[:, 1], hit_positions[:, 2],
        hit_primIDs,
        ibo[:, 0], ibo[:, 1], ibo[:, 2],
        vbo[:, 0], vbo[:, 1], vbo[:, 2],
        features)


# speedup vs baseline: 2.8846x; 1.0588x over previous
import functools

import jax
import jax.numpy as jnp
from jax import lax
from jax.experimental import pallas as pl
from jax.experimental.pallas import tpu as pltpu
from jax.experimental.pallas import tpu_sc as plsc

N_HITS = 1048576
N_VERTS = 262144
N_TRIS = 524288
N_PARAMS = 16

_info = plsc.get_sparse_core_info()
_NC, _NS, _L = _info.num_cores, _info.num_subcores, _info.num_lanes
_NW = _NC * _NS

CHUNK = 512
HITS_PER_W = N_HITS // _NW
NCH = HITS_PER_W // CHUNK
GROUPS = CHUNK // _L
_SL = 128
_NSL = CHUNK // _SL


def _body(hx_hbm, hy_hbm, hz_hbm, prim_hbm,
          i0_hbm, i1_hbm, i2_hbm, vx_hbm, vy_hbm, vz_hbm, feat_hbm,
          out_hbm,
          prim_v, idx_v, hp_v, vc_v, f0_v, f1_v, f2_v, out_v,
          sem_t, sem_b):
    wid = lax.axis_index("s") * _NC + lax.axis_index("c")
    wbase = wid * HITS_PER_W
    ibufs = (i0_hbm, i1_hbm, i2_hbm)
    vbufs = (vx_hbm, vy_hbm, vz_hbm)
    hbufs = (hx_hbm, hy_hbm, hz_hbm)
    fbufs = (f0_v, f1_v, f2_v)

    def tri_descs(n, r, mk):
        del n
        op = pltpu.make_async_copy if mk else pltpu.async_copy
        return [
            op(ibufs[k].at[prim_v.at[r, pl.ds(s * _SL, _SL)]],
               idx_v.at[r, k, pl.ds(s * _SL, _SL)], sem_t)
            for k in range(3)
            for s in range(_NSL)
        ]

    def prim_load(n, r):
        base = wbase + n * CHUNK
        pltpu.sync_copy(prim_hbm.at[pl.ds(base, CHUNK)], prim_v.at[r])

    def bulk_descs(n, r, mk):
        base = wbase + n * CHUNK
        op = pltpu.make_async_copy if mk else pltpu.async_copy
        cps = [
            op(hbufs[d].at[pl.ds(base, CHUNK)], hp_v.at[r, d], sem_b)
            for d in range(3)
        ]
        for k in range(3):
            for s in range(_NSL):
                isl = idx_v.at[r, k, pl.ds(s * _SL, _SL)]
                sl = pl.ds(s * _SL, _SL)
                cps += [
                    op(vbufs[d].at[isl], vc_v.at[r, 3 * k + d, sl], sem_b)
                    for d in range(3)
                ]
                cps.append(op(feat_hbm.at[isl], fbufs[k].at[r, sl], sem_b))
        return cps

    def compute(n, r):
        def grp(j, c):
            b = j * _L
            sl = pl.ds(b, _L)
            hx, hy, hz = hp_v[r, 0, sl], hp_v[r, 1, sl], hp_v[r, 2, sl]
            ax, ay, az = vc_v[r, 0, sl], vc_v[r, 1, sl], vc_v[r, 2, sl]
            bx, by, bz = vc_v[r, 3, sl], vc_v[r, 4, sl], vc_v[r, 5, sl]
            cx, cy, cz = vc_v[r, 6, sl], vc_v[r, 7, sl], vc_v[r, 8, sl]

            e0x, e0y, e0z = bx - ax, by - ay, bz - az
            e1x, e1y, e1z = cx - ax, cy - ay, cz - az
            px, py, pz = hx - ax, hy - ay, hz - az

            d00 = e0x * e0x + e0y * e0y + e0z * e0z
            d01 = e0x * e1x + e0y * e1y + e0z * e1z
            d11 = e1x * e1x + e1y * e1y + e1z * e1z
            d20 = px * e0x + py * e0y + pz * e0z
            d21 = px * e1x + py * e1y + pz * e1z

            denom = d00 * d11 - d01 * d01
            denom = jnp.where(jnp.abs(denom) < 1e-12,
                              jnp.float32(1e-12), denom)
            vv = (d11 * d20 - d01 * d21) / denom
            ww = (d00 * d21 - d01 * d20) / denom
            uu = 1.0 - vv - ww

            for i in range(_L):
                out_v[b + i, :] = (uu[i] * f0_v[r, b + i, :]
                                   + vv[i] * f1_v[r, b + i, :]
                                   + ww[i] * f2_v[r, b + i, :])
            return c
        lax.fori_loop(0, GROUPS, grp, 0)
        base = wbase + n * CHUNK
        pltpu.sync_copy(out_v, out_hbm.at[pl.ds(base, CHUNK)])

    prim_load(0, 0)
    tri_descs(0, 0, False)
    for d in tri_descs(0, 0, True):
        d.wait()
    bulk_descs(0, 0, False)
    prim_load(1, 1)
    tri_descs(1, 1, False)

    def step(g, p):
        q = 1 - p
        for d in bulk_descs(g, p, True):
            d.wait()

        @pl.when(g < NCH - 1)
        def _():
            for d in tri_descs(g + 1, q, True):
                d.wait()
            bulk_descs(g + 1, q, False)

        @pl.when(g < NCH - 2)
        def _():
            prim_load(g + 2, p)
            tri_descs(g + 2, p, False)

        compute(g, p)

    def chunk_body(m, carry):
        step(2 * m, 0)
        step(2 * m + 1, 1)
        return carry

    lax.fori_loop(0, NCH // 2, chunk_body, 0)


_sc_kernel = functools.partial(
    pl.kernel,
    out_type=jax.ShapeDtypeStruct((N_HITS, N_PARAMS), jnp.float32),
    mesh=plsc.VectorSubcoreMesh(core_axis_name="c", subcore_axis_name="s"),
    scratch_types=[
        pltpu.VMEM((2, CHUNK), jnp.int32),
        pltpu.VMEM((2, 3, CHUNK), jnp.int32),
        pltpu.VMEM((2, 3, CHUNK), jnp.float32),
        pltpu.VMEM((2, 9, CHUNK), jnp.float32),
        pltpu.VMEM((2, CHUNK, N_PARAMS), jnp.float32),
        pltpu.VMEM((2, CHUNK, N_PARAMS), jnp.float32),
        pltpu.VMEM((2, CHUNK, N_PARAMS), jnp.float32),
        pltpu.VMEM((CHUNK, N_PARAMS), jnp.float32),
        pltpu.SemaphoreType.DMA,
        pltpu.SemaphoreType.DMA,
    ],
    compiler_params=pltpu.CompilerParams(
        use_tc_tiling_on_sc=False,
        needs_layout_passes=False,
    ),
)(_body)


@jax.jit
def kernel(hit_positions, hit_primIDs, vbo, ibo, features):
    return _sc_kernel(
        hit_positions[:, 0], hit_positions[:, 1], hit_positions[:, 2],
        hit_primIDs,
        ibo[:, 0], ibo[:, 1], ibo[:, 2],
        vbo[:, 0], vbo[:, 1], vbo[:, 2],
        features)
